# Initial kernel scaffold; baseline (speedup 1.0000x reference)
import jax
import jax.numpy as jnp
from jax.experimental import pallas as pl


def kernel(sub_id, neigh_orders, w_sub, b_sub, w_fc, b_fc, g0, be0, w0, b0, g1, be1, w1, b1, g2, be2, w2, b2):
    raise NotImplementedError("write your pallas kernel here")


# speedup vs baseline: 1.0085x; 1.0085x over previous
import functools

import jax
import jax.numpy as jnp
from jax.experimental import pallas as pl
from jax.experimental.pallas import tpu as pltpu

_COL_TILE = 2048


def _round_up(x, mult):
    return ((x + mult - 1) // mult) * mult


def _head_kernel(sub_ref, wsub_ref, bsub_ref, wfc_ref, bfc_ref,
                 o_ref, st_ref, *, total, cin):
    j = pl.program_id(0)
    hidden = (jnp.dot(sub_ref[...], wsub_ref[...],
                      preferred_element_type=jnp.float32) + bsub_ref[...])
    o = (jnp.dot(hidden, wfc_ref[...],
                 preferred_element_type=jnp.float32) + bfc_ref[...])
    o_ref[...] = o

    tn = o.shape[1]
    col = j * tn + jax.lax.broadcasted_iota(jnp.int32, (1, tn), 1)
    ch = col % cin
    valid = col < total
    rid = jax.lax.broadcasted_iota(jnp.int32, (8, 8), 0)
    cid = jax.lax.broadcasted_iota(jnp.int32, (8, 8), 1)
    acc = jnp.zeros((8, 8), jnp.float32)
    for c in range(cin):
        oc = jnp.where(valid & (ch == c), o, 0.0)
        s = jnp.sum(oc)
        q = jnp.sum(oc * oc)
        acc = jnp.where((rid == 0) & (cid == c), s, acc)
        acc = jnp.where((rid == 1) & (cid == c), q, acc)
    st_ref[...] = acc[None]


def _fc_head(sub_id, w_sub, b_sub, w_fc, b_fc, cin):
    _, n_sub = sub_id.shape
    hid = w_sub.shape[1]
    total = w_fc.shape[1]
    tn = min(total, _COL_TILE)
    nt = pl.cdiv(total, tn)
    kern = functools.partial(_head_kernel, total=total, cin=cin)
    return pl.pallas_call(
        kern,
        out_shape=(jax.ShapeDtypeStruct((1, total), jnp.float32),
                   jax.ShapeDtypeStruct((nt, 8, 8), jnp.float32)),
        grid=(nt,),
        in_specs=[
            pl.BlockSpec((1, n_sub), lambda j: (0, 0)),
            pl.BlockSpec((n_sub, hid), lambda j: (0, 0)),
            pl.BlockSpec((1, hid), lambda j: (0, 0)),
            pl.BlockSpec((hid, tn), lambda j: (0, j)),
            pl.BlockSpec((1, tn), lambda j: (0, j)),
        ],
        out_specs=(pl.BlockSpec((1, tn), lambda j: (0, j)),
                   pl.BlockSpec((1, 8, 8), lambda j: (j, 0, 0))),
        compiler_params=pltpu.CompilerParams(
            dimension_semantics=("parallel",)),
    )(sub_id, w_sub, b_sub, w_fc, b_fc)


def _conv_kernel(gh_ref, st_ref, g_ref, be_ref, w_ref, b_ref,
                 out_ref, so_ref, *, rows, cin, slope, eps, emit_stats):
    st = st_ref[...]
    s = jnp.sum(st[:, 0, :], axis=0, keepdims=True)[:, :cin]
    q = jnp.sum(st[:, 1, :], axis=0, keepdims=True)[:, :cin]
    mean = s / rows
    var = q / rows - mean * mean
    inv = jax.lax.rsqrt(var + eps)
    scale = g_ref[...] * inv
    shift = be_ref[...] - mean * scale
    scale7 = jnp.concatenate([scale] * 7, axis=1)
    shift7 = jnp.concatenate([shift] * 7, axis=1)

    z = gh_ref[...] * scale7 + shift7
    y = jnp.where(z >= 0, z, slope * z)
    o = (jnp.dot(y, w_ref[...], preferred_element_type=jnp.float32)
         + b_ref[...])
    out_ref[...] = o

    if emit_stats:
        tm, cout = o.shape
        row = (pl.program_id(0) * tm
               + jax.lax.broadcasted_iota(jnp.int32, (tm, 1), 0))
        om = jnp.where(row < rows, o, 0.0)
        so = jnp.sum(om, axis=0, keepdims=True)
        qo = jnp.sum(om * om, axis=0, keepdims=True)
        rid = jax.lax.broadcasted_iota(jnp.int32, (8, cout), 0)
        so_ref[...] = jnp.where(rid == 0, so,
                                jnp.where(rid == 1, qo, 0.0))[None]


def _conv_layer(gh, stats, gamma, beta, w, b, m, *,
                emit_stats, eps=1e-5, slope=0.2):
    k, cout = w.shape
    cin = k // 7
    p, _, c_st = stats.shape
    nt = 2
    tm = _round_up(pl.cdiv(m, nt), 8)
    kern = functools.partial(_conv_kernel, rows=m, cin=cin,
                             slope=slope, eps=eps, emit_stats=emit_stats)
    out_shapes = (jax.ShapeDtypeStruct((m, cout), jnp.float32),
                  jax.ShapeDtypeStruct((nt, 8, cout), jnp.float32))
    out_specs = (pl.BlockSpec((tm, cout), lambda i: (i, 0)),
                 pl.BlockSpec((1, 8, cout), lambda i: (i, 0, 0)))
    out, st_out = pl.pallas_call(
        kern,
        out_shape=out_shapes,
        grid=(nt,),
        in_specs=[
            pl.BlockSpec((tm, k), lambda i: (i, 0)),
            pl.BlockSpec((p, 8, c_st), lambda i: (0, 0, 0)),
            pl.BlockSpec((1, cin), lambda i: (0, 0)),
            pl.BlockSpec((1, cin), lambda i: (0, 0)),
            pl.BlockSpec((k, cout), lambda i: (0, 0)),
            pl.BlockSpec((1, cout), lambda i: (0, 0)),
        ],
        out_specs=out_specs,
        compiler_params=pltpu.CompilerParams(
            dimension_semantics=("parallel",)),
    )(gh, stats, gamma.reshape(1, cin), beta.reshape(1, cin), w, b)
    return out, st_out


def kernel(sub_id, neigh_orders, w_sub, b_sub, w_fc, b_fc,
           g0, be0, w0, b0, g1, be1, w1, b1, g2, be2, w2, b2):
    cin0 = 3
    n_vertex = w_fc.shape[1] // cin0

    x_flat, st = _fc_head(sub_id, w_sub, b_sub, w_fc, b_fc, cin0)
    x = x_flat.reshape(n_vertex, cin0)

    gh0 = jnp.take(x, neigh_orders, axis=0).reshape(n_vertex, 7 * cin0)
    x1, st1 = _conv_layer(gh0, st, g0, be0, w0, b0, n_vertex,
                          emit_stats=True)

    gh1 = jnp.take(x1, neigh_orders, axis=0).reshape(n_vertex,
                                                     7 * w1.shape[0] // 7)
    x2, st2 = _conv_layer(gh1, st1, g1, be1, w1, b1, n_vertex,
                          emit_stats=True)

    gh2 = jnp.take(x2, neigh_orders, axis=0).reshape(n_vertex, w2.shape[0])
    x3, _ = _conv_layer(gh2, st2, g2, be2, w2, b2, n_vertex,
                        emit_stats=False)
    return x3


# speedup vs baseline: 1.9139x; 1.8979x over previous
import functools

import jax
import jax.numpy as jnp
from jax.experimental import pallas as pl
from jax.experimental.pallas import tpu as pltpu


def _round_up(x, mult):
    return ((x + mult - 1) // mult) * mult


def _head_kernel(sub_ref, wsubt_ref, wfc_ref, o_ref, *, ns):
    s = pl.program_id(1)
    h = jax.lax.dot_general(
        sub_ref[...], wsubt_ref[...],
        (((1,), (1,)), ((), ())),
        preferred_element_type=jnp.float32)
    p = jnp.dot(h, wfc_ref[...], preferred_element_type=jnp.float32)

    @pl.when(s == 0)
    def _():
        o_ref[0] = p

    @pl.when(s > 0)
    def _():
        o_ref[0] += p


def _fc_head(sub_id, w_sub, b_sub, w_fc):
    _, n_sub = sub_id.shape
    hid = w_sub.shape[1]
    total = w_fc.shape[1]
    ns = 4 if hid % 8 == 0 and (hid // 2) % 4 == 0 else 1
    kr = hid // 2 // ns
    sub_aug = jnp.concatenate(
        [sub_id, jnp.ones((1, 1), sub_id.dtype)], axis=1)
    wsub_aug_t = jnp.concatenate([w_sub, b_sub], axis=0).T
    kern = functools.partial(_head_kernel, ns=ns)
    return pl.pallas_call(
        kern,
        out_shape=jax.ShapeDtypeStruct((2, 1, total), jnp.float32),
        grid=(2, ns),
        in_specs=[
            pl.BlockSpec((1, n_sub + 1), lambda c, s: (0, 0)),
            pl.BlockSpec((kr, n_sub + 1), lambda c, s: (c * ns + s, 0)),
            pl.BlockSpec((kr, total), lambda c, s: (c * ns + s, 0)),
        ],
        out_specs=pl.BlockSpec((1, 1, total), lambda c, s: (c, 0, 0)),
        compiler_params=pltpu.CompilerParams(
            dimension_semantics=("parallel", "arbitrary")),
    )(sub_aug, wsub_aug_t, w_fc)


_UNROLL = 8
_CHUNK = 192


def _gconv_kernel(x_ref, idx_ref, g_ref, be_ref, w_ref, b_ref, out_ref,
                  z3_ref, o3_ref, *, m, cin, cout, tm, slope, eps):
    x = x_ref[...]
    s = jnp.sum(x, axis=0, keepdims=True)
    q = jnp.sum(x * x, axis=0, keepdims=True)
    mean = s / m
    var = q / m - mean * mean
    inv = jax.lax.rsqrt(var + eps)
    scale = g_ref[...] * inv
    shift = be_ref[...] - mean * scale
    z = x * scale + shift
    y = jnp.where(z >= 0, z, slope * z)

    wstack = jnp.concatenate(
        [w_ref[cin * j:cin * (j + 1), :] for j in range(7)]
        + [jnp.zeros((cin, cout), jnp.float32)], axis=1)
    zz = jnp.dot(y, wstack, preferred_element_type=jnp.float32)
    gw = 8 * cout
    z3_ref[...] = zz.reshape(m, 1, gw)

    lane = jax.lax.broadcasted_iota(jnp.int32, (1, gw), 1)
    masks = [(lane // cout) == j for j in range(7)]
    rid = jax.lax.broadcasted_iota(jnp.int32, (gw, cout), 0)
    cid = jax.lax.broadcasted_iota(jnp.int32, (gw, cout), 1)
    sel_r = ((rid % cout) == cid).astype(jnp.float32)
    bias = b_ref[...]

    def outer(ko, carry):
        tb = ko * (7 * _UNROLL)
        for ui in range(_UNROLL):
            rows = [z3_ref[idx_ref[0, 0, tb + 7 * ui + j]] for j in range(7)]
            sel = [jnp.where(masks[j], rows[j], 0.0) for j in range(7)]
            o3_ref[ko * _UNROLL + ui] = (((sel[0] + sel[1])
                                          + (sel[2] + sel[3]))
                                         + ((sel[4] + sel[5]) + sel[6]))
        return carry

    jax.lax.fori_loop(0, tm // _UNROLL, outer, 0)

    def red(kc, carry):
        blk = o3_ref[pl.ds(kc * _CHUNK, _CHUNK)]
        t2 = blk.reshape(_CHUNK, gw)
        o = jnp.dot(t2, sel_r, preferred_element_type=jnp.float32)
        out_ref[pl.ds(kc * _CHUNK, _CHUNK), :] = o + bias
        return carry

    jax.lax.fori_loop(0, tm // _CHUNK, red, 0)


def _gconv_layer(x, idx2, gamma, beta, w, b, *, slope=0.2, eps=1e-5):
    m, cin = x.shape
    cout = w.shape[1]
    tm = idx2.shape[2] // 7
    kern = functools.partial(_gconv_kernel, m=m, cin=cin, cout=cout,
                             tm=tm, slope=slope, eps=eps)
    return pl.pallas_call(
        kern,
        out_shape=jax.ShapeDtypeStruct((m, cout), jnp.float32),
        grid=(2,),
        in_specs=[
            pl.BlockSpec((m, cin), lambda c: (0, 0)),
            pl.BlockSpec((1, 1, idx2.shape[2]), lambda c: (c, 0, 0),
                         memory_space=pltpu.SMEM),
            pl.BlockSpec((1, cin), lambda c: (0, 0)),
            pl.BlockSpec((1, cin), lambda c: (0, 0)),
            pl.BlockSpec(w.shape, lambda c: (0, 0)),
            pl.BlockSpec((1, cout), lambda c: (0, 0)),
        ],
        out_specs=pl.BlockSpec((tm, cout), lambda c: (c, 0)),
        scratch_shapes=[
            pltpu.VMEM((m, 1, 8 * cout), jnp.float32),
            pltpu.VMEM((tm, 1, 8 * cout), jnp.float32),
        ],
        compiler_params=pltpu.CompilerParams(
            dimension_semantics=("parallel",)),
    )(x, idx2, gamma.reshape(1, cin), beta.reshape(1, cin), w, b)


def kernel(sub_id, neigh_orders, w_sub, b_sub, w_fc, b_fc,
           g0, be0, w0, b0, g1, be1, w1, b1, g2, be2, w2, b2):
    cin0 = 3
    n_vertex = w_fc.shape[1] // cin0

    parts = _fc_head(sub_id, w_sub, b_sub, w_fc)
    x = (parts[0, 0] + parts[1, 0] + b_fc[0]).reshape(n_vertex, cin0)

    tm = _round_up((n_vertex + 1) // 2, _CHUNK)
    idx2 = jnp.pad(neigh_orders, (0, 2 * 7 * tm - neigh_orders.shape[0])
                   ).reshape(2, 1, 7 * tm)

    x1 = _gconv_layer(x, idx2, g0, be0, w0, b0)
    x2 = _gconv_layer(x1, idx2, g1, be1, w1, b1)
    x3 = _gconv_layer(x2, idx2, g2, be2, w2, b2)
    return x3


# speedup vs baseline: 2.0027x; 1.0464x over previous
import functools

import jax
import jax.numpy as jnp
from jax.experimental import pallas as pl
from jax.experimental.pallas import tpu as pltpu


def _round_up(x, mult):
    return ((x + mult - 1) // mult) * mult


def _head_kernel(sub_ref, wsubt_ref, wfc_ref, o_ref, *, ns):
    s = pl.program_id(1)
    h = jax.lax.dot_general(
        sub_ref[...], wsubt_ref[...],
        (((1,), (1,)), ((), ())),
        preferred_element_type=jnp.float32)
    p = jnp.dot(h, wfc_ref[...], preferred_element_type=jnp.float32)

    @pl.when(s == 0)
    def _():
        o_ref[0] = p

    @pl.when(s > 0)
    def _():
        o_ref[0] += p


def _fc_head(sub_id, w_sub, b_sub, w_fc):
    _, n_sub = sub_id.shape
    hid = w_sub.shape[1]
    total = w_fc.shape[1]
    ns = 4 if hid % 8 == 0 and (hid // 2) % 4 == 0 else 1
    kr = hid // 2 // ns
    sub_aug = jnp.concatenate(
        [sub_id, jnp.ones((1, 1), sub_id.dtype)], axis=1)
    wsub_aug_t = jnp.concatenate([w_sub, b_sub], axis=0).T
    kern = functools.partial(_head_kernel, ns=ns)
    return pl.pallas_call(
        kern,
        out_shape=jax.ShapeDtypeStruct((2, 1, total), jnp.float32),
        grid=(2, ns),
        in_specs=[
            pl.BlockSpec((1, n_sub + 1), lambda c, s: (0, 0)),
            pl.BlockSpec((kr, n_sub + 1), lambda c, s: (c * ns + s, 0)),
            pl.BlockSpec((kr, total), lambda c, s: (c * ns + s, 0)),
        ],
        out_specs=pl.BlockSpec((1, 1, total), lambda c, s: (c, 0, 0)),
        compiler_params=pltpu.CompilerParams(
            dimension_semantics=("parallel", "arbitrary")),
    )(sub_aug, wsub_aug_t, w_fc)


_UNROLL = 16
_CHUNK = 192


def _gconv_kernel(x_ref, idx_ref, g_ref, be_ref, w_ref, b_ref, out_ref,
                  z3_ref, o3_ref, *, m, cin, cout, tm, slope, eps):
    x = x_ref[...]
    s = jnp.sum(x, axis=0, keepdims=True)
    q = jnp.sum(x * x, axis=0, keepdims=True)
    mean = s / m
    var = q / m - mean * mean
    inv = jax.lax.rsqrt(var + eps)
    scale = g_ref[...] * inv
    shift = be_ref[...] - mean * scale
    z = x * scale + shift
    y = jnp.where(z >= 0, z, slope * z)

    wstack = jnp.concatenate(
        [w_ref[cin * j:cin * (j + 1), :] for j in range(7)]
        + [jnp.zeros((cin, cout), jnp.float32)], axis=1)
    zz = jnp.dot(y, wstack, preferred_element_type=jnp.float32)
    gw = 8 * cout
    z3_ref[...] = zz.reshape(m, 1, gw)

    lane = jax.lax.broadcasted_iota(jnp.int32, (1, gw), 1)
    masks = [(lane // cout) == j for j in range(7)]
    rid = jax.lax.broadcasted_iota(jnp.int32, (gw, cout), 0)
    cid = jax.lax.broadcasted_iota(jnp.int32, (gw, cout), 1)
    sel_r = ((rid % cout) == cid).astype(jnp.float32)
    bias = b_ref[...]

    def outer(ko, carry):
        tb = ko * (7 * _UNROLL)
        for ui in range(_UNROLL):
            rows = [z3_ref[idx_ref[0, 0, tb + 7 * ui + j]] for j in range(7)]
            sel = [jnp.where(masks[j], rows[j], 0.0) for j in range(7)]
            o3_ref[ko * _UNROLL + ui] = (((sel[0] + sel[1])
                                          + (sel[2] + sel[3]))
                                         + ((sel[4] + sel[5]) + sel[6]))
        return carry

    jax.lax.fori_loop(0, tm // _UNROLL, outer, 0)

    def red(kc, carry):
        blk = o3_ref[pl.ds(kc * _CHUNK, _CHUNK)]
        t2 = blk.reshape(_CHUNK, gw)
        o = jnp.dot(t2, sel_r, preferred_element_type=jnp.float32)
        out_ref[pl.ds(kc * _CHUNK, _CHUNK), :] = o + bias
        return carry

    jax.lax.fori_loop(0, tm // _CHUNK, red, 0)


def _gconv_layer(x, idx2, gamma, beta, w, b, *, slope=0.2, eps=1e-5):
    m, cin = x.shape
    cout = w.shape[1]
    tm = idx2.shape[2] // 7
    kern = functools.partial(_gconv_kernel, m=m, cin=cin, cout=cout,
                             tm=tm, slope=slope, eps=eps)
    return pl.pallas_call(
        kern,
        out_shape=jax.ShapeDtypeStruct((m, cout), jnp.float32),
        grid=(2,),
        in_specs=[
            pl.BlockSpec((m, cin), lambda c: (0, 0)),
            pl.BlockSpec((1, 1, idx2.shape[2]), lambda c: (c, 0, 0),
                         memory_space=pltpu.SMEM),
            pl.BlockSpec((1, cin), lambda c: (0, 0)),
            pl.BlockSpec((1, cin), lambda c: (0, 0)),
            pl.BlockSpec(w.shape, lambda c: (0, 0)),
            pl.BlockSpec((1, cout), lambda c: (0, 0)),
        ],
        out_specs=pl.BlockSpec((tm, cout), lambda c: (c, 0)),
        scratch_shapes=[
            pltpu.VMEM((m, 1, 8 * cout), jnp.float32),
            pltpu.VMEM((tm, 1, 8 * cout), jnp.float32),
        ],
        compiler_params=pltpu.CompilerParams(
            dimension_semantics=("parallel",)),
    )(x, idx2, gamma.reshape(1, cin), beta.reshape(1, cin), w, b)


def kernel(sub_id, neigh_orders, w_sub, b_sub, w_fc, b_fc,
           g0, be0, w0, b0, g1, be1, w1, b1, g2, be2, w2, b2):
    cin0 = 3
    n_vertex = w_fc.shape[1] // cin0

    parts = _fc_head(sub_id, w_sub, b_sub, w_fc)
    x = (parts[0, 0] + parts[1, 0] + b_fc[0]).reshape(n_vertex, cin0)

    tm = _round_up((n_vertex + 1) // 2, _CHUNK)
    idx2 = jnp.pad(neigh_orders, (0, 2 * 7 * tm - neigh_orders.shape[0])
                   ).reshape(2, 1, 7 * tm)

    x1 = _gconv_layer(x, idx2, g0, be0, w0, b0)
    x2 = _gconv_layer(x1, idx2, g1, be1, w1, b1)
    x3 = _gconv_layer(x2, idx2, g2, be2, w2, b2)
    return x3
